# Initial kernel scaffold; baseline (speedup 1.0000x reference)
import jax
import jax.numpy as jnp
from jax.experimental import pallas as pl


def kernel(TeacherLogit, Target, W, Iscale):
    raise NotImplementedError("write your pallas kernel here")


# speedup vs baseline: 1.3266x; 1.3266x over previous
import jax
import jax.numpy as jnp
from jax.experimental import pallas as pl
from jax.experimental.pallas import tpu as pltpu

NCLASS = 100
DIM = 128
BATCH = 4096


def _rout_kernel(isc_ref, tl_ref, tgt_ref, w_ref, out_ref):
    c = pl.program_id(0)
    w0 = w_ref[0, 0]
    w1 = w_ref[0, 1]
    a = jax.lax.dot_general(
        w0, w1, (((1,), (1,)), ((), ())), preferred_element_type=jnp.float32
    )
    isc = isc_ref[c]
    rows = jax.lax.broadcasted_iota(jnp.int32, (NCLASS, NCLASS), 0)
    cols = jax.lax.broadcasted_iota(jnp.int32, (NCLASS, NCLASS), 1)
    a = a + jnp.where(rows == cols, isc, jnp.float32(0.0))
    amax = jnp.max(a, axis=0, keepdims=True)
    e = jnp.exp(a - amax)
    m_c = e / jnp.sum(e, axis=0, keepdims=True)

    p = tl_ref[...]
    pm = jnp.dot(p, m_c, preferred_element_type=jnp.float32)
    mask = tgt_ref[...] == c

    @pl.when(c == 0)
    def _():
        out_ref[...] = jnp.zeros_like(out_ref)

    acc = jnp.where(mask, pm, out_ref[...])

    @pl.when(c < NCLASS - 1)
    def _():
        out_ref[...] = acc

    @pl.when(c == NCLASS - 1)
    def _():
        s = jnp.sum(p, axis=1, keepdims=True)
        out_ref[...] = acc / s


def kernel(TeacherLogit, Target, W, Iscale):
    tgt = Target.reshape(BATCH, 1)
    out = pl.pallas_call(
        _rout_kernel,
        grid=(NCLASS,),
        in_specs=[
            pl.BlockSpec(memory_space=pltpu.SMEM),
            pl.BlockSpec((BATCH, NCLASS), lambda c: (0, 0)),
            pl.BlockSpec((BATCH, 1), lambda c: (0, 0)),
            pl.BlockSpec((1, 2, NCLASS, DIM), lambda c: (c, 0, 0, 0)),
        ],
        out_specs=pl.BlockSpec((BATCH, NCLASS), lambda c: (0, 0)),
        out_shape=jax.ShapeDtypeStruct((BATCH, NCLASS), jnp.float32),
    )(Iscale, TeacherLogit, tgt, W)
    return out


# speedup vs baseline: 1.5019x; 1.1322x over previous
import functools

import jax
import jax.numpy as jnp
from jax import lax
from jax.experimental import pallas as pl
from jax.experimental.pallas import tpu as pltpu
from jax.experimental.pallas import tpu_sc as plsc

NCLASS = 100
DIM = 128
BATCH = 4096
NPAD = 128

_NC = 2
_NS = 16
_NW = _NC * _NS
_BPW = BATCH // _NW


def _sc_gather_body(table_hbm, idx_hbm, out_hbm, idx_v, rows_v, sem):
    wid = lax.axis_index("s") * _NC + lax.axis_index("c")
    base = wid * _BPW
    pltpu.sync_copy(idx_hbm.at[pl.ds(base, _BPW)], idx_v)
    pltpu.async_copy(table_hbm.at[idx_v], rows_v, sem).wait()
    pltpu.sync_copy(rows_v, out_hbm.at[pl.ds(base, _BPW)])


def _sc_gather(table, idx):
    mesh = plsc.VectorSubcoreMesh(
        core_axis_name="c", subcore_axis_name="s",
        num_cores=_NC, num_subcores=_NS,
    )
    return pl.kernel(
        _sc_gather_body,
        out_type=jax.ShapeDtypeStruct((BATCH, NPAD), jnp.float32),
        mesh=mesh,
        scratch_types=[
            pltpu.VMEM((_BPW,), jnp.int32),
            pltpu.VMEM((_BPW, NPAD), jnp.float32),
            pltpu.SemaphoreType.DMA,
        ],
    )(table, idx)


def _m_kernel(isc_ref, w_ref, m_ref):
    c = pl.program_id(0)
    w0 = w_ref[0, 0]
    w1 = w_ref[0, 1]
    a = lax.dot_general(
        w0, w1, (((1,), (1,)), ((), ())), preferred_element_type=jnp.float32
    )
    rows = lax.broadcasted_iota(jnp.int32, (NCLASS, NCLASS), 0)
    cols = lax.broadcasted_iota(jnp.int32, (NCLASS, NCLASS), 1)
    a = a + jnp.where(rows == cols, isc_ref[c], jnp.float32(0.0))
    amax = jnp.max(a, axis=0, keepdims=True)
    e = jnp.exp(a - amax)
    m = e / jnp.sum(e, axis=0, keepdims=True)
    m_ref[0] = jnp.pad(m, ((0, NPAD - NCLASS), (0, NPAD - NCLASS)))


def _rout_kernel(first_ref, last_ref, tsort_ref, st_ref, m_ref, out_ref):
    j = pl.program_id(0)
    p = tsort_ref[...]
    lo = first_ref[j]
    hi = last_ref[j]

    def body(c, acc):
        m_c = m_ref[c]
        pm = jnp.dot(p, m_c, preferred_element_type=jnp.float32)
        mask = st_ref[...] == c
        return jnp.where(mask, pm, acc)

    acc = lax.fori_loop(lo, hi + 1, body, jnp.zeros_like(p))
    s = jnp.sum(p, axis=1, keepdims=True)
    out_ref[...] = acc / s


_ROWS = 128
_NBLK = BATCH // _ROWS


def kernel(TeacherLogit, Target, W, Iscale):
    tl_pad = jnp.pad(TeacherLogit, ((0, 0), (0, NPAD - NCLASS)))
    iota = lax.iota(jnp.int32, BATCH)
    st, order = lax.sort((Target, iota), num_keys=1)
    inv = jnp.zeros((BATCH,), jnp.int32).at[order].set(iota)
    st_blocks = st.reshape(_NBLK, _ROWS)
    first = st_blocks[:, 0]
    last = st_blocks[:, -1]

    tsort = _sc_gather(tl_pad, order)

    m_pad = pl.pallas_call(
        _m_kernel,
        grid=(NCLASS,),
        in_specs=[
            pl.BlockSpec(memory_space=pltpu.SMEM),
            pl.BlockSpec((1, 2, NCLASS, DIM), lambda c: (c, 0, 0, 0)),
        ],
        out_specs=pl.BlockSpec((1, NPAD, NPAD), lambda c: (c, 0, 0)),
        out_shape=jax.ShapeDtypeStruct((NCLASS, NPAD, NPAD), jnp.float32),
    )(Iscale, W)

    out_sorted = pl.pallas_call(
        _rout_kernel,
        grid=(_NBLK,),
        in_specs=[
            pl.BlockSpec(memory_space=pltpu.SMEM),
            pl.BlockSpec(memory_space=pltpu.SMEM),
            pl.BlockSpec((_ROWS, NPAD), lambda j: (j, 0)),
            pl.BlockSpec((_ROWS, 1), lambda j: (j, 0)),
            pl.BlockSpec((NCLASS, NPAD, NPAD), lambda j: (0, 0, 0)),
        ],
        out_specs=pl.BlockSpec((_ROWS, NPAD), lambda j: (j, 0)),
        out_shape=jax.ShapeDtypeStruct((BATCH, NPAD), jnp.float32),
    )(first, last, tsort, st.reshape(BATCH, 1), m_pad)

    out = _sc_gather(out_sorted, inv)
    return out[:, :NCLASS]


# speedup vs baseline: 2.3263x; 1.5489x over previous
import jax
import jax.numpy as jnp
from jax import lax
from jax.experimental import pallas as pl
from jax.experimental.pallas import tpu as pltpu
from jax.experimental.pallas import tpu_sc as plsc

NCLASS = 100
DIM = 128
BATCH = 4096
NPAD = 128

_NC = 2
_NS = 16
_NW = _NC * _NS
_BPW = BATCH // _NW

_CB = 10
_MSTEPS = NCLASS // _CB
_ROWS = 128
_NBLK = BATCH // _ROWS


def _sc_gather_body(table_hbm, idx_hbm, out_hbm, idx_v, rows_v, sem):
    wid = lax.axis_index("s") * _NC + lax.axis_index("c")
    base = wid * _BPW
    pltpu.sync_copy(idx_hbm.at[pl.ds(base, _BPW)], idx_v)
    pltpu.async_copy(table_hbm.at[idx_v], rows_v, sem).wait()
    pltpu.sync_copy(rows_v, out_hbm.at[pl.ds(base, _BPW)])


def _sc_gather(table, idx):
    mesh = plsc.VectorSubcoreMesh(
        core_axis_name="c", subcore_axis_name="s",
        num_cores=_NC, num_subcores=_NS,
    )
    return pl.kernel(
        _sc_gather_body,
        out_type=jax.ShapeDtypeStruct((BATCH, NPAD), jnp.float32),
        mesh=mesh,
        scratch_types=[
            pltpu.VMEM((_BPW,), jnp.int32),
            pltpu.VMEM((_BPW, NPAD), jnp.float32),
            pltpu.SemaphoreType.DMA,
        ],
    )(table, idx)


def _fused_kernel(isc_ref, first_ref, last_ref, w_ref, tsort_ref, st_ref,
                  out_ref, m_ref):
    i = pl.program_id(0)

    @pl.when(i < _MSTEPS)
    def _():
        for k in range(_CB):
            c = i * _CB + k
            w0 = w_ref[k, 0]
            w1 = w_ref[k, 1]
            a = lax.dot_general(
                w0, w1, (((1,), (1,)), ((), ())),
                preferred_element_type=jnp.float32,
            )
            rows = lax.broadcasted_iota(jnp.int32, (NCLASS, NCLASS), 0)
            cols = lax.broadcasted_iota(jnp.int32, (NCLASS, NCLASS), 1)
            a = a + jnp.where(rows == cols, isc_ref[c], jnp.float32(0.0))
            amax = jnp.max(a, axis=0, keepdims=True)
            e = jnp.exp(a - amax)
            m = e / jnp.sum(e, axis=0, keepdims=True)
            m_ref[c] = jnp.pad(m, ((0, NPAD - NCLASS), (0, NPAD - NCLASS)))

    @pl.when(i >= _MSTEPS)
    def _():
        j = i - _MSTEPS
        p = tsort_ref[...]
        lo = first_ref[j]
        hi = last_ref[j]

        def body(c, acc):
            m_c = m_ref[c]
            pm = jnp.dot(p, m_c, preferred_element_type=jnp.float32)
            mask = st_ref[...] == c
            return jnp.where(mask, pm, acc)

        acc = lax.fori_loop(lo, hi + 1, body, jnp.zeros_like(p))
        s = jnp.sum(p, axis=1, keepdims=True)
        out_ref[...] = acc / s


def kernel(TeacherLogit, Target, W, Iscale):
    tl_pad = jnp.pad(TeacherLogit, ((0, 0), (0, NPAD - NCLASS)))
    iota = lax.iota(jnp.int32, BATCH)
    st, order = lax.sort((Target, iota), num_keys=1)
    inv = jnp.zeros((BATCH,), jnp.int32).at[order].set(iota)
    st_blocks = st.reshape(_NBLK, _ROWS)
    first = st_blocks[:, 0]
    last = st_blocks[:, -1]

    tsort = _sc_gather(tl_pad, order)

    out_sorted = pl.pallas_call(
        _fused_kernel,
        grid=(_MSTEPS + _NBLK,),
        in_specs=[
            pl.BlockSpec(memory_space=pltpu.SMEM),
            pl.BlockSpec(memory_space=pltpu.SMEM),
            pl.BlockSpec(memory_space=pltpu.SMEM),
            pl.BlockSpec((_CB, 2, NCLASS, DIM),
                         lambda i: (jnp.minimum(i, _MSTEPS - 1), 0, 0, 0)),
            pl.BlockSpec((_ROWS, NPAD),
                         lambda i: (jnp.clip(i - _MSTEPS, 0, _NBLK - 1), 0)),
            pl.BlockSpec((_ROWS, 1),
                         lambda i: (jnp.clip(i - _MSTEPS, 0, _NBLK - 1), 0)),
        ],
        out_specs=pl.BlockSpec(
            (_ROWS, NPAD), lambda i: (jnp.clip(i - _MSTEPS, 0, _NBLK - 1), 0)
        ),
        out_shape=jax.ShapeDtypeStruct((BATCH, NPAD), jnp.float32),
        scratch_shapes=[pltpu.VMEM((NCLASS, NPAD, NPAD), jnp.float32)],
    )(Iscale, first, last, W, tsort, st.reshape(BATCH, 1))

    out = _sc_gather(out_sorted, inv)
    return out[:, :NCLASS]
